# Initial kernel scaffold; baseline (speedup 1.0000x reference)
import jax
import jax.numpy as jnp
from jax.experimental import pallas as pl


def kernel(node_table, edge_table, node_inputs, edge_inputs):
    raise NotImplementedError("write your pallas kernel here")


# speedup vs baseline: 1.6329x; 1.6329x over previous
import functools
import jax
import jax.numpy as jnp
from jax import lax
from jax.experimental import pallas as pl
from jax.experimental.pallas import tpu as pltpu
from jax.experimental.pallas import tpu_sc as plsc

EMB = 64
B = 4096 * 50
NC = 2
NS = 16
NW = NC * NS
BPW = B // NW
CH = 640
NCHUNK = BPW // CH

_mesh = plsc.VectorSubcoreMesh(core_axis_name="c", subcore_axis_name="s")


@functools.partial(
    pl.kernel,
    mesh=_mesh,
    out_type=[
        jax.ShapeDtypeStruct((B, EMB), jnp.float32),
        jax.ShapeDtypeStruct((B, EMB), jnp.float32),
    ],
    scratch_types=[
        pltpu.VMEM((CH,), jnp.int32),
        pltpu.VMEM((CH, EMB), jnp.float32),
        pltpu.SemaphoreType.DMA,
    ],
    compiler_params=pltpu.CompilerParams(use_tc_tiling_on_sc=False),
)
def _gather2(node_table, edge_table, node_idx, edge_idx,
             node_out, edge_out, idx_v, rows_v, sem):
    wid = lax.axis_index("s") * NC + lax.axis_index("c")
    base = wid * BPW

    def run(table_hbm, idx_hbm, out_hbm):
        def chunk(j, carry):
            off = base + j * CH
            pltpu.sync_copy(idx_hbm.at[pl.ds(off, CH)], idx_v)
            pltpu.async_copy(table_hbm.at[idx_v], rows_v, sem).wait()
            pltpu.sync_copy(rows_v, out_hbm.at[pl.ds(off, CH)])
            return carry
        lax.fori_loop(0, NCHUNK, chunk, 0)

    run(node_table, node_idx, node_out)
    run(edge_table, edge_idx, edge_out)


def kernel(node_table, edge_table, node_inputs, edge_inputs):
    bshape = node_inputs.shape
    n_idx = node_inputs.reshape(-1).astype(jnp.int32)
    e_idx = edge_inputs.reshape(-1).astype(jnp.int32)
    node_out, edge_out = _gather2(node_table, edge_table, n_idx, e_idx)
    return (node_out.reshape(*bshape, EMB), edge_out.reshape(*bshape, EMB))


# speedup vs baseline: 1.6676x; 1.0213x over previous
import functools
import jax
import jax.numpy as jnp
from jax import lax
from jax.experimental import pallas as pl
from jax.experimental.pallas import tpu as pltpu
from jax.experimental.pallas import tpu_sc as plsc

EMB = 64
B = 4096 * 50
NC = 2
NS = 16
NW = NC * NS
BPW = B // NW
CH = 800
NCHUNK = BPW // CH
NBUF = 2

_mesh = plsc.VectorSubcoreMesh(core_axis_name="c", subcore_axis_name="s")


@functools.partial(
    pl.kernel,
    mesh=_mesh,
    out_type=[
        jax.ShapeDtypeStruct((B, EMB), jnp.float32),
        jax.ShapeDtypeStruct((B, EMB), jnp.float32),
    ],
    scratch_types=[
        pltpu.VMEM((BPW,), jnp.int32),
        pltpu.VMEM((BPW,), jnp.int32),
        pltpu.VMEM((NBUF, CH, EMB), jnp.float32),
        pltpu.SemaphoreType.DMA((NBUF,)),
    ],
    compiler_params=pltpu.CompilerParams(use_tc_tiling_on_sc=False),
)
def _gather2(node_table, edge_table, node_idx, edge_idx,
             node_out, edge_out, nidx_v, eidx_v, rows_v, sems):
    wid = lax.axis_index("s") * NC + lax.axis_index("c")
    base = wid * BPW

    pltpu.sync_copy(node_idx.at[pl.ds(base, BPW)], nidx_v)
    pltpu.sync_copy(edge_idx.at[pl.ds(base, BPW)], eidx_v)

    chunks = [(node_table, nidx_v, node_out, j * CH) for j in range(NCHUNK)]
    chunks += [(edge_table, eidx_v, edge_out, j * CH) for j in range(NCHUNK)]

    pending = []
    for k, (tab, idx_v, out_hbm, off) in enumerate(chunks):
        b = k % NBUF
        if len(pending) == NBUF:
            cd, p_out, p_off, p_b = pending.pop(0)
            cd.wait()
            pltpu.sync_copy(rows_v.at[p_b], p_out.at[pl.ds(base + p_off, CH)])
        cd = pltpu.async_copy(tab.at[idx_v.at[pl.ds(off, CH)]],
                              rows_v.at[b], sems.at[b])
        pending.append((cd, out_hbm, off, b))
    for cd, p_out, p_off, p_b in pending:
        cd.wait()
        pltpu.sync_copy(rows_v.at[p_b], p_out.at[pl.ds(base + p_off, CH)])


def kernel(node_table, edge_table, node_inputs, edge_inputs):
    bshape = node_inputs.shape
    n_idx = node_inputs.reshape(-1).astype(jnp.int32)
    e_idx = edge_inputs.reshape(-1).astype(jnp.int32)
    node_out, edge_out = _gather2(node_table, edge_table, n_idx, e_idx)
    return (node_out.reshape(*bshape, EMB), edge_out.reshape(*bshape, EMB))


# speedup vs baseline: 1.7131x; 1.0273x over previous
import functools
import jax
import jax.numpy as jnp
from jax import lax
from jax.experimental import pallas as pl
from jax.experimental.pallas import tpu as pltpu
from jax.experimental.pallas import tpu_sc as plsc

EMB = 64
B = 4096 * 50
NC = 2
NS = 16
NW = NC * NS
BPW = B // NW
CH = 800
NCHUNK = BPW // CH
NBUF = 2

_mesh = plsc.VectorSubcoreMesh(core_axis_name="c", subcore_axis_name="s")


def _make_gather():
    @functools.partial(
        pl.kernel,
        mesh=_mesh,
        out_type=jax.ShapeDtypeStruct((B, EMB), jnp.float32),
        scratch_types=[
            pltpu.VMEM((BPW,), jnp.int32),
            pltpu.VMEM((NBUF, CH, EMB), jnp.float32),
            pltpu.SemaphoreType.DMA((NBUF,)),
        ],
        compiler_params=pltpu.CompilerParams(use_tc_tiling_on_sc=False),
    )
    def _g(table, idx, out, idx_v, rows_v, sems):
        wid = lax.axis_index("s") * NC + lax.axis_index("c")
        base = wid * BPW
        pltpu.sync_copy(idx.at[pl.ds(base, BPW)], idx_v)
        pending = []
        for j in range(NCHUNK):
            b = j % NBUF
            if len(pending) == NBUF:
                cd, p_j, p_b = pending.pop(0)
                cd.wait()
                pltpu.sync_copy(rows_v.at[p_b],
                                out.at[pl.ds(base + p_j * CH, CH)])
            cd = pltpu.async_copy(table.at[idx_v.at[pl.ds(j * CH, CH)]],
                                  rows_v.at[b], sems.at[b])
            pending.append((cd, j, b))
        for cd, p_j, p_b in pending:
            cd.wait()
            pltpu.sync_copy(rows_v.at[p_b],
                            out.at[pl.ds(base + p_j * CH, CH)])
    return _g


_gather_node = _make_gather()
_gather_edge = _make_gather()


def kernel(node_table, edge_table, node_inputs, edge_inputs):
    bshape = node_inputs.shape
    n_idx = node_inputs.reshape(-1).astype(jnp.int32)
    e_idx = edge_inputs.reshape(-1).astype(jnp.int32)
    node_out = _gather_node(node_table, n_idx)
    edge_out = _gather_edge(edge_table, e_idx)
    return (node_out.reshape(*bshape, EMB), edge_out.reshape(*bshape, EMB))
